# baseline (speedup 1.0000x reference)
import functools

import jax
import jax.numpy as jnp
import numpy as np
from jax.experimental import pallas as pl
from jax.experimental.pallas import tpu as pltpu
from jax.experimental.pallas import tpu_sc as plsc

_BINS = 10
_BLOCK_R = 512

_THRESH = [
    np.float32(np.log(e / (1.0 - e)))
    for e in (np.float64(np.float32(k) / np.float32(_BINS)) for k in range(1, _BINS))
]


def _ghm_kernel(x_ref, v_ref, out_ref, acc_ref):
    i = pl.program_id(0)
    nb = pl.num_programs(0)

    @pl.when(i == 0)
    def _init():
        for b in range(_BINS):
            acc_ref[0, b] = 0.0
            acc_ref[1, b] = 0.0

    x = x_ref[...]
    rows = x.shape[0]
    bce = jnp.maximum(x, 0.0) + jnp.log1p(jnp.exp(-jnp.abs(x)))
    bce_bf = bce.astype(jnp.bfloat16)

    lhs = jnp.full((8, rows), 0.125, dtype=jnp.bfloat16)

    def colsum(mat_bf):
        res = jax.lax.dot_general(
            lhs, mat_bf, (((1,), (0,)), ((), ())),
            preferred_element_type=jnp.float32)
        return jnp.sum(res)

    v = v_ref[...]
    bce0v = jnp.maximum(v, 0.0) + jnp.log1p(jnp.exp(-jnp.abs(v)))

    acc_ref[0, 0] += np.float32(x.size)
    acc_ref[1, 0] += colsum(bce_bf) - jnp.sum(v)
    for k in range(1, _BINS):
        thr = _THRESH[k - 1]
        m = x >= thr
        mp = v >= thr
        mn = v <= -thr
        acc_ref[0, k] += (colsum(m.astype(jnp.bfloat16))
                          + jnp.sum(jnp.where(mn, 1.0, 0.0))
                          - jnp.sum(jnp.where(mp, 1.0, 0.0)))
        acc_ref[1, k] += (colsum(jnp.where(m, bce_bf, jnp.bfloat16(0)))
                          + jnp.sum(jnp.where(mn, bce0v - v, 0.0))
                          - jnp.sum(jnp.where(mp, bce0v, 0.0)))

    @pl.when(i == nb - 1)
    def _finish():
        n = jnp.float32(0.0)
        total = jnp.float32(0.0)
        for b in range(_BINS):
            if b < _BINS - 1:
                cb = acc_ref[0, b] - acc_ref[0, b + 1]
                sb = acc_ref[1, b] - acc_ref[1, b + 1]
            else:
                cb = acc_ref[0, b]
                sb = acc_ref[1, b]
            nonempty = cb > 0.0
            n = n + jnp.where(nonempty, 1.0, 0.0)
            total = total + jnp.where(nonempty, sb / jnp.maximum(cb, 1.0), 0.0)
        out_ref[0, 0] = total / jnp.maximum(n, 1.0)


def _sc_gather(xflat, addr):
    info = plsc.get_sparse_core_info()
    nc, ns = info.num_cores, info.num_subcores
    nw = nc * ns
    b = addr.shape[0]
    b_per_w = b // nw
    mesh = plsc.VectorSubcoreMesh(core_axis_name="c", subcore_axis_name="s")

    @functools.partial(
        pl.kernel, mesh=mesh,
        out_type=jax.ShapeDtypeStruct((b,), jnp.float32),
        scratch_types=[
            pltpu.VMEM((b_per_w,), jnp.int32),
            pltpu.VMEM((b_per_w,), jnp.float32),
            pltpu.SemaphoreType.DMA,
        ],
    )
    def gather_k(xflat_hbm, addr_hbm, out_hbm, idx_v, val_v, sem):
        wid = jax.lax.axis_index("s") * nc + jax.lax.axis_index("c")
        base = wid * b_per_w
        pltpu.sync_copy(addr_hbm.at[pl.ds(base, b_per_w)], idx_v)
        pltpu.async_copy(xflat_hbm.at[idx_v], val_v, sem).wait()
        pltpu.sync_copy(val_v, out_hbm.at[pl.ds(base, b_per_w)])

    return gather_k(xflat, addr)


def kernel(input, target):
    rows, cols = input.shape
    addr = jnp.arange(rows, dtype=jnp.int32) * cols + target.astype(jnp.int32)
    xtv = _sc_gather(input.reshape(-1), addr)

    block_r = min(_BLOCK_R, rows)
    grid = rows // block_r
    out = pl.pallas_call(
        _ghm_kernel,
        grid=(grid,),
        in_specs=[
            pl.BlockSpec((block_r, cols), lambda i: (i, 0)),
            pl.BlockSpec((block_r,), lambda i: (i,)),
        ],
        out_specs=pl.BlockSpec(memory_space=pltpu.SMEM),
        out_shape=jax.ShapeDtypeStruct((1, 1), jnp.float32),
        scratch_shapes=[pltpu.SMEM((2, _BINS), jnp.float32)],
    )(input, xtv)
    return out[0, 0]


# speedup vs baseline: 1.4640x; 1.4640x over previous
import jax
import jax.numpy as jnp
import numpy as np
from jax.experimental import pallas as pl
from jax.experimental.pallas import tpu as pltpu

_BINS = 10
_BLOCK_R = 1024

_THRESH = [
    np.float32(np.log(e / (1.0 - e)))
    for e in (np.float64(np.float32(k) / np.float32(_BINS)) for k in range(1, _BINS))
]


def _ghm_kernel(x_ref, t_ref, out_ref, acc_ref):
    i = pl.program_id(0)
    nb = pl.num_programs(0)

    @pl.when(i == 0)
    def _init():
        for b in range(_BINS):
            acc_ref[0, b] = 0.0
            acc_ref[1, b] = 0.0

    x = x_ref[...]
    t2 = t_ref[...]
    rows = x.shape[0]
    cols = jax.lax.broadcasted_iota(jnp.int32, x.shape, 1)
    is_t = cols == t2
    nx = -x
    xt = jnp.where(is_t, nx, x)
    bce = jnp.maximum(x, 0.0) + 0.5 * (xt - x) + jnp.log(1.0 + jnp.exp(jnp.minimum(x, nx)))
    bce_bf = bce.astype(jnp.bfloat16)

    lhs = jnp.full((8, rows), 0.125, dtype=jnp.bfloat16)

    def colsum(mat_bf):
        res = jax.lax.dot_general(
            lhs, mat_bf, (((1,), (0,)), ((), ())),
            preferred_element_type=jnp.float32)
        return jnp.sum(res)

    acc_ref[0, 0] += np.float32(x.size)
    acc_ref[1, 0] += colsum(bce_bf)
    for k in range(1, _BINS):
        m = xt >= _THRESH[k - 1]
        acc_ref[0, k] += colsum(jnp.where(m, 1.0, 0.0).astype(jnp.bfloat16))
        acc_ref[1, k] += colsum(jnp.where(m, bce_bf, jnp.bfloat16(0)))

    @pl.when(i == nb - 1)
    def _finish():
        n = jnp.float32(0.0)
        total = jnp.float32(0.0)
        for b in range(_BINS):
            if b < _BINS - 1:
                cb = acc_ref[0, b] - acc_ref[0, b + 1]
                sb = acc_ref[1, b] - acc_ref[1, b + 1]
            else:
                cb = acc_ref[0, b]
                sb = acc_ref[1, b]
            nonempty = cb > 0.0
            n = n + jnp.where(nonempty, 1.0, 0.0)
            total = total + jnp.where(nonempty, sb / jnp.maximum(cb, 1.0), 0.0)
        out_ref[0, 0] = total / jnp.maximum(n, 1.0)


def kernel(input, target):
    rows, cols = input.shape
    block_r = min(_BLOCK_R, rows)
    grid = rows // block_r
    out = pl.pallas_call(
        _ghm_kernel,
        grid=(grid,),
        in_specs=[
            pl.BlockSpec((block_r, cols), lambda i: (i, 0)),
            pl.BlockSpec((block_r, 1), lambda i: (i, 0)),
        ],
        out_specs=pl.BlockSpec(memory_space=pltpu.SMEM),
        out_shape=jax.ShapeDtypeStruct((1, 1), jnp.float32),
        scratch_shapes=[pltpu.SMEM((2, _BINS), jnp.float32)],
    )(input, target.astype(jnp.int32).reshape(rows, 1))
    return out[0, 0]


# speedup vs baseline: 1.4829x; 1.0129x over previous
import jax
import jax.numpy as jnp
import numpy as np
from jax.experimental import pallas as pl
from jax.experimental.pallas import tpu as pltpu

_BINS = 10
_BLOCK_R = 2048

_THRESH = [
    np.float32(np.log(e / (1.0 - e)))
    for e in (np.float64(np.float32(k) / np.float32(_BINS)) for k in range(1, _BINS))
]


def _ghm_kernel(x_ref, t_ref, out_ref, acc_ref):
    i = pl.program_id(0)
    nb = pl.num_programs(0)

    @pl.when(i == 0)
    def _init():
        for b in range(_BINS):
            acc_ref[0, b] = 0.0
            acc_ref[1, b] = 0.0

    x = x_ref[...]
    t2 = t_ref[...]
    rows = x.shape[0]
    cols = jax.lax.broadcasted_iota(jnp.int32, x.shape, 1)
    is_t = cols == t2
    nx = -x
    xt = jnp.where(is_t, nx, x)
    bce = jnp.maximum(x, 0.0) + 0.5 * (xt - x) + jnp.log(1.0 + jnp.exp(jnp.minimum(x, nx)))
    bce_bf = bce.astype(jnp.bfloat16)

    lhs = jnp.full((8, rows), 0.125, dtype=jnp.bfloat16)

    def colsum(mat_bf):
        res = jax.lax.dot_general(
            lhs, mat_bf, (((1,), (0,)), ((), ())),
            preferred_element_type=jnp.float32)
        return jnp.sum(res)

    acc_ref[0, 0] += np.float32(x.size)
    acc_ref[1, 0] += colsum(bce_bf)
    for k in range(1, _BINS):
        m = xt >= _THRESH[k - 1]
        acc_ref[0, k] += colsum(jnp.where(m, 1.0, 0.0).astype(jnp.bfloat16))
        acc_ref[1, k] += colsum(jnp.where(m, bce_bf, jnp.bfloat16(0)))

    @pl.when(i == nb - 1)
    def _finish():
        n = jnp.float32(0.0)
        total = jnp.float32(0.0)
        for b in range(_BINS):
            if b < _BINS - 1:
                cb = acc_ref[0, b] - acc_ref[0, b + 1]
                sb = acc_ref[1, b] - acc_ref[1, b + 1]
            else:
                cb = acc_ref[0, b]
                sb = acc_ref[1, b]
            nonempty = cb > 0.0
            n = n + jnp.where(nonempty, 1.0, 0.0)
            total = total + jnp.where(nonempty, sb / jnp.maximum(cb, 1.0), 0.0)
        out_ref[0, 0] = total / jnp.maximum(n, 1.0)


def kernel(input, target):
    rows, cols = input.shape
    block_r = min(_BLOCK_R, rows)
    grid = rows // block_r
    out = pl.pallas_call(
        _ghm_kernel,
        grid=(grid,),
        in_specs=[
            pl.BlockSpec((block_r, cols), lambda i: (i, 0)),
            pl.BlockSpec((block_r, 1), lambda i: (i, 0)),
        ],
        out_specs=pl.BlockSpec(memory_space=pltpu.SMEM),
        out_shape=jax.ShapeDtypeStruct((1, 1), jnp.float32),
        scratch_shapes=[pltpu.SMEM((2, _BINS), jnp.float32)],
    )(input, target.astype(jnp.int32).reshape(rows, 1))
    return out[0, 0]
